# baseline (speedup 1.0000x reference)
import functools
import math

import jax
import jax.numpy as jnp
from jax.experimental import pallas as pl
from jax.experimental.pallas import tpu as pltpu
from jax.experimental.pallas import tpu_sc as plsc

_SC_W = 128


def _sc_mesh():
    return plsc.VectorSubcoreMesh(core_axis_name="c", subcore_axis_name="s")

_EPS = 1e-6
_NEG = -1e30


def _dot_t(a, b):
    return jax.lax.dot_general(a, b, (((1,), (1,)), ((), ())),
                               preferred_element_type=jnp.float32)


def _dot3(a, b):
    return jnp.dot(a, b, preferred_element_type=jnp.float32)


def _rms(xf, w):
    ms = jnp.mean(xf * xf, axis=-1, keepdims=True)
    return xf / jnp.sqrt(ms + _EPS) * w


def _qkv_body(x_ref, nw_ref, w_ref, o_ref):
    xn = _rms(x_ref[...], nw_ref[...])
    o_ref[...] = _dot_t(xn, w_ref[...])


def _attn_body(qkv_ref, o_ref, *, bq, hd, nh, d):
    i = pl.program_id(0)
    rows = i * bq + jax.lax.broadcasted_iota(jnp.int32, (bq, bq), 0)
    for h in range(nh):
        q = qkv_ref[pl.ds(i * bq, bq), h * hd:(h + 1) * hd]

        def inner(j, carry):
            m, l, acc = carry
            k = qkv_ref[pl.ds(j * bq, bq), d + h * hd:d + (h + 1) * hd]
            v = qkv_ref[pl.ds(j * bq, bq), 2 * d + h * hd:2 * d + (h + 1) * hd]
            s = _dot_t(q, k) * (1.0 / math.sqrt(hd))
            cols = j * bq + jax.lax.broadcasted_iota(jnp.int32, (bq, bq), 1)
            s = jnp.where(cols <= rows, s, _NEG)
            mj = jnp.max(s, axis=-1, keepdims=True)
            m_new = jnp.maximum(m, mj)
            p = jnp.exp(s - m_new)
            scale = jnp.exp(m - m_new)
            l_new = l * scale + jnp.sum(p, axis=-1, keepdims=True)
            acc_new = acc * scale + _dot3(p, v)
            return m_new, l_new, acc_new

        m0 = jnp.full((bq, 1), _NEG, jnp.float32)
        l0 = jnp.zeros((bq, 1), jnp.float32)
        a0 = jnp.zeros((bq, hd), jnp.float32)
        m, l, acc = jax.lax.fori_loop(0, i + 1, inner, (m0, l0, a0))
        o_ref[:, h * hd:(h + 1) * hd] = acc / l


def _proj_body(attn_ref, x_ref, ow_ref, nw_ref, wsg_ref, wsu_ref, wsd_ref,
               xres_ref, xn2_ref, shared_ref):
    a = _dot_t(attn_ref[...], ow_ref[...])
    xr = x_ref[...] + a
    xres_ref[...] = xr
    xn = _rms(xr, nw_ref[...])
    xnb = xn.astype(jnp.bfloat16)
    xn2_ref[...] = xnb
    g = jnp.dot(xnb, wsg_ref[...], preferred_element_type=jnp.float32)
    u = jnp.dot(xnb, wsu_ref[...], preferred_element_type=jnp.float32)
    hs = (g * jax.nn.sigmoid(g) * u).astype(jnp.bfloat16)
    shared_ref[...] = jnp.dot(hs, wsd_ref[...], preferred_element_type=jnp.float32)


def _cumsum0(x, n):
    s = 1
    while s < n:
        x = x + jnp.concatenate(
            [jnp.zeros((s, x.shape[1]), x.dtype), x[:-s]], axis=0)
        s *= 2
    return x


def _tri_lt(n):
    ii = jax.lax.broadcasted_iota(jnp.int32, (n, n), 0)
    jj = jax.lax.broadcasted_iota(jnp.int32, (n, n), 1)
    return (ii < jj).astype(jnp.float32)


def _router_body(xres_ref, nw_ref, rwt_ref, wts_ref, pos_ref, gid_ref,
                 aux_ref, *, ne, coeff, bm, cap, nb, t):
    xn = _rms(xres_ref[...], nw_ref[...])
    logits = _dot_t(xn, rwt_ref[...])
    lm = jnp.max(logits, axis=-1, keepdims=True)
    ex = jnp.exp(logits - lm)
    probs = ex / jnp.sum(ex, axis=-1, keepdims=True)
    idx = jax.lax.broadcasted_iota(jnp.int32, probs.shape, 1)
    m1 = jnp.max(probs, axis=-1, keepdims=True)
    i1 = jnp.min(jnp.where(probs == m1, idx, ne), axis=-1, keepdims=True)
    oh1 = (idx == i1)
    p2 = jnp.where(oh1, _NEG, probs)
    m2 = jnp.max(p2, axis=-1, keepdims=True)
    i2 = jnp.min(jnp.where(p2 == m2, idx, ne), axis=-1, keepdims=True)
    oh2 = (idx == i2)
    denom = m1 + m2
    wts_ref[...] = jnp.concatenate([m1 / denom, m2 / denom], axis=1)
    frac = jnp.mean((oh1 | oh2).astype(jnp.float32), axis=0, keepdims=True)
    pmean = jnp.mean(probs, axis=0, keepdims=True)
    aux_ref[...] = jnp.sum(frac * pmean).reshape(1, 1) * (coeff * ne)

    o1 = oh1.astype(jnp.float32)
    o2 = oh2.astype(jnp.float32)
    c1 = _cumsum0(o1, t)
    c2 = _cumsum0(o2, t)
    cnt1 = c1[t - 1:t, :]
    cnt = cnt1 + c2[t - 1:t, :]
    pcnt = jnp.floor((cnt + (bm - 1)) * (1.0 / bm)) * bm
    offs = jnp.dot(pcnt, _tri_lt(ne), preferred_element_type=jnp.float32)
    rank0 = jnp.sum(o1 * (c1 - o1), axis=1, keepdims=True)
    rank1 = jnp.sum(o2 * (cnt1 + c2 - o2), axis=1, keepdims=True)
    base0 = jnp.sum(o1 * offs, axis=1, keepdims=True)
    base1 = jnp.sum(o2 * offs, axis=1, keepdims=True)
    pos0 = base0 + rank0
    pos1 = base1 + rank1

    tot = offs[:, ne - 1:ne] + pcnt[:, ne - 1:ne]
    starts9 = jnp.concatenate([offs + cnt, tot], axis=1)
    plens9 = jnp.concatenate([pcnt - cnt, cap - tot], axis=1)
    cl9 = jnp.dot(plens9, _tri_lt(ne + 1), preferred_element_type=jnp.float32)
    npad = cap - 2 * t
    j = jax.lax.broadcasted_iota(jnp.int32, (npad, ne + 1),
                                 0).astype(jnp.float32)
    inr = ((j >= cl9) & (j < cl9 + plens9)).astype(jnp.float32)
    padpos = jnp.sum(inr * (starts9 + j - cl9), axis=1, keepdims=True)

    pos_ref[...] = jnp.concatenate([pos0, pos1, padpos],
                                   axis=0).astype(jnp.int32)
    bidx = jax.lax.broadcasted_iota(jnp.int32, (nb, ne),
                                    0).astype(jnp.float32) * bm
    gid_ref[...] = (jnp.sum((bidx >= offs).astype(jnp.float32), axis=1,
                            keepdims=True) - 1.0).astype(jnp.int32)


def _sc_scatter(vals, idx, cap):
    @pl.kernel(out_type=jax.ShapeDtypeStruct((cap, 128), jnp.int32),
               mesh=_sc_mesh(), scratch_types=[])
    def k(v_hbm, i_hbm, o_hbm):
        def body(v_vmem, i_vmem):
            pltpu.sync_copy(v_vmem, o_hbm.at[i_vmem.at[0]])

        pltpu.emit_pipeline(
            body,
            grid=(cap // _SC_W,),
            in_specs=[pl.BlockSpec((_SC_W, 128), lambda i: (i, 0)),
                      pl.BlockSpec((1, _SC_W), lambda i: (0, i))],
            out_specs=[],
            core_axis_name=("c", "s"),
            dimension_semantics=(pltpu.PARALLEL,),
        )(v_hbm, i_hbm)

    return k(vals, idx)


def _sc_gather(data, idx, n, width, dtype, w=32):
    @pl.kernel(out_type=jax.ShapeDtypeStruct((n, width), dtype),
               mesh=_sc_mesh(), scratch_types=[])
    def k(x_hbm, i_hbm, o_hbm):
        def body(i_vmem, o_vmem):
            pltpu.sync_copy(x_hbm.at[i_vmem.at[0]], o_vmem)

        pltpu.emit_pipeline(
            body,
            grid=(n // w,),
            in_specs=[pl.BlockSpec((1, w), lambda i: (0, i))],
            out_specs=[pl.BlockSpec((w, width), lambda i: (i, 0))],
            core_axis_name=("c", "s"),
            dimension_semantics=(pltpu.PARALLEL,),
        )(i_hbm, o_hbm)

    return k(data, idx)


def _gmm_body(gid_ref, xin_ref, wg_ref, wu_ref, wd_ref, ye_ref):
    del gid_ref
    xb = xin_ref[...]
    g = jnp.dot(xb, wg_ref[0], preferred_element_type=jnp.float32)
    u = jnp.dot(xb, wu_ref[0], preferred_element_type=jnp.float32)
    hh = (g * jax.nn.sigmoid(g) * u).astype(jnp.bfloat16)
    ye_ref[...] = jnp.dot(hh, wd_ref[0],
                          preferred_element_type=jnp.float32).astype(
                              jnp.bfloat16)


def _combine_body(xres_ref, shared_ref, y0_ref, y1_ref, wts_ref, o_ref):
    w = wts_ref[...]
    o_ref[...] = (xres_ref[...] + shared_ref[...]
                  + w[:, 0:1] * y0_ref[...].astype(jnp.float32)
                  + w[:, 1:2] * y1_ref[...].astype(jnp.float32))


def kernel(x, attn_norm_w, qkv_w, out_w, ffn_norm_w, router_w, w_gate, w_up,
           w_down, ws_gate, ws_up, ws_down):
    B, T, D = x.shape
    E, _, F = w_gate.shape
    H = 16
    hd = D // H
    BT = min(256, T)
    BM = min(512, T)
    x2 = x.reshape(T, D)
    bf = jnp.bfloat16

    anw = attn_norm_w.reshape(1, D)
    fnw = ffn_norm_w.reshape(1, D)

    qkv = pl.pallas_call(
        _qkv_body,
        grid=(T // BT,),
        in_specs=[
            pl.BlockSpec((BT, D), lambda i: (i, 0)),
            pl.BlockSpec((1, D), lambda i: (0, 0)),
            pl.BlockSpec((3 * D, D), lambda i: (0, 0)),
        ],
        out_specs=pl.BlockSpec((BT, 3 * D), lambda i: (i, 0)),
        out_shape=jax.ShapeDtypeStruct((T, 3 * D), jnp.float32),
        compiler_params=pltpu.CompilerParams(
            dimension_semantics=("arbitrary",)),
    )(x2, anw, qkv_w)

    BQ = min(512, T)
    attn = pl.pallas_call(
        functools.partial(_attn_body, bq=BQ, hd=hd, nh=H, d=D),
        grid=(T // BQ,),
        in_specs=[
            pl.BlockSpec((T, 3 * D), lambda i: (0, 0)),
        ],
        out_specs=pl.BlockSpec((BQ, D), lambda i: (i, 0)),
        out_shape=jax.ShapeDtypeStruct((T, D), jnp.float32),
        compiler_params=pltpu.CompilerParams(
            dimension_semantics=("arbitrary",)),
    )(qkv)

    xres, xn2, shared = pl.pallas_call(
        _proj_body,
        grid=(T // BT,),
        in_specs=[
            pl.BlockSpec((BT, D), lambda i: (i, 0)),
            pl.BlockSpec((BT, D), lambda i: (i, 0)),
            pl.BlockSpec((D, D), lambda i: (0, 0)),
            pl.BlockSpec((1, D), lambda i: (0, 0)),
            pl.BlockSpec((D, F), lambda i: (0, 0)),
            pl.BlockSpec((D, F), lambda i: (0, 0)),
            pl.BlockSpec((F, D), lambda i: (0, 0)),
        ],
        out_specs=[
            pl.BlockSpec((BT, D), lambda i: (i, 0)),
            pl.BlockSpec((BT, D), lambda i: (i, 0)),
            pl.BlockSpec((BT, D), lambda i: (i, 0)),
        ],
        out_shape=[
            jax.ShapeDtypeStruct((T, D), jnp.float32),
            jax.ShapeDtypeStruct((T, D), bf),
            jax.ShapeDtypeStruct((T, D), jnp.float32),
        ],
        compiler_params=pltpu.CompilerParams(
            dimension_semantics=("arbitrary",)),
    )(attn, x2, out_w, fnw, ws_gate.astype(bf), ws_up.astype(bf),
      ws_down.astype(bf))

    BE = min(256, T)
    NB = -(-(2 * T + E * (BE - 1)) // BE)
    CAP = NB * BE

    wts, posall, gid, aux = pl.pallas_call(
        functools.partial(_router_body, ne=E, coeff=0.01, bm=BE, cap=CAP,
                          nb=NB, t=T),
        grid=(1,),
        in_specs=[
            pl.BlockSpec((T, D), lambda i: (0, 0)),
            pl.BlockSpec((1, D), lambda i: (0, 0)),
            pl.BlockSpec((E, D), lambda i: (0, 0)),
        ],
        out_specs=[
            pl.BlockSpec((T, 2), lambda i: (0, 0)),
            pl.BlockSpec((CAP, 1), lambda i: (0, 0)),
            pl.BlockSpec((NB, 1), lambda i: (0, 0)),
            pl.BlockSpec((1, 1), lambda i: (0, 0)),
        ],
        out_shape=[
            jax.ShapeDtypeStruct((T, 2), jnp.float32),
            jax.ShapeDtypeStruct((CAP, 1), jnp.int32),
            jax.ShapeDtypeStruct((NB, 1), jnp.int32),
            jax.ShapeDtypeStruct((1, 1), jnp.float32),
        ],
    )(xres, fnw, router_w)

    ar = jnp.arange(T, dtype=jnp.int32)
    tokvals = jnp.broadcast_to(
        jnp.concatenate([ar, ar, jnp.zeros(CAP - 2 * T, jnp.int32)])[:, None],
        (CAP, 128))
    idx_all = posall.reshape(1, CAP)
    sidx = _sc_scatter(tokvals, idx_all, CAP)[:, 0:1]
    l4 = jnp.arange(4, dtype=jnp.int32)[None, :]
    xn2p = jax.lax.bitcast_convert_type(
        xn2.reshape(T, D // 2, 2), jnp.int32).reshape(4 * T, D // 8)
    sidx4 = (sidx * 4 + l4).reshape(1, 4 * CAP)
    xinp = _sc_gather(xn2p, sidx4, 4 * CAP, D // 8, jnp.int32, w=_SC_W)
    xin = jax.lax.bitcast_convert_type(
        xinp.reshape(CAP, D // 2), bf).reshape(CAP, D)

    ye = pl.pallas_call(
        _gmm_body,
        grid_spec=pltpu.PrefetchScalarGridSpec(
            num_scalar_prefetch=1,
            grid=(NB,),
            in_specs=[
                pl.BlockSpec((BE, D), lambda b, g: (b, 0)),
                pl.BlockSpec((1, D, F), lambda b, g: (g[b], 0, 0)),
                pl.BlockSpec((1, D, F), lambda b, g: (g[b], 0, 0)),
                pl.BlockSpec((1, F, D), lambda b, g: (g[b], 0, 0)),
            ],
            out_specs=pl.BlockSpec((BE, D), lambda b, g: (b, 0)),
        ),
        out_shape=jax.ShapeDtypeStruct((CAP, D), bf),
        compiler_params=pltpu.CompilerParams(
            dimension_semantics=("arbitrary",)),
    )(gid.reshape(NB), xin, w_gate.astype(bf), w_up.astype(bf),
      w_down.astype(bf))

    yep = jax.lax.bitcast_convert_type(
        ye.reshape(CAP, D // 2, 2), jnp.int32).reshape(4 * CAP, D // 8)
    pos4 = (posall[:2 * T] * 4 + l4).reshape(1, 8 * T)
    ykp = _sc_gather(yep, pos4, 8 * T, D // 8, jnp.int32, w=_SC_W)
    yk = jax.lax.bitcast_convert_type(
        ykp.reshape(2 * T, D // 2), bf).reshape(2 * T, D)

    BC = min(512, T)
    NC = T // BC
    y = pl.pallas_call(
        _combine_body,
        grid=(T // BC,),
        in_specs=[
            pl.BlockSpec((BC, D), lambda t: (t, 0)),
            pl.BlockSpec((BC, D), lambda t: (t, 0)),
            pl.BlockSpec((BC, D), lambda t: (t, 0)),
            pl.BlockSpec((BC, D), lambda t: (t + NC, 0)),
            pl.BlockSpec((BC, 2), lambda t: (t, 0)),
        ],
        out_specs=pl.BlockSpec((BC, D), lambda t: (t, 0)),
        out_shape=jax.ShapeDtypeStruct((T, D), jnp.float32),
        compiler_params=pltpu.CompilerParams(
            dimension_semantics=("arbitrary",)),
    )(xres, shared, yk, yk, wts)

    return (y.reshape(B, T, D), aux[0, 0])


# speedup vs baseline: 2.8715x; 2.8715x over previous
import functools
import math

import jax
import jax.numpy as jnp
from jax.experimental import pallas as pl
from jax.experimental.pallas import tpu as pltpu

_EPS = 1e-6
_NEG = -1e30


def _dot_t(a, b):
    return jax.lax.dot_general(a, b, (((1,), (1,)), ((), ())),
                               preferred_element_type=jnp.float32)


def _dot3(a, b):
    return jnp.dot(a, b, preferred_element_type=jnp.float32)


def _rms(xf, w):
    ms = jnp.mean(xf * xf, axis=-1, keepdims=True)
    return xf / jnp.sqrt(ms + _EPS) * w


def _qkv_body(x_ref, nw_ref, w_ref, o_ref):
    xn = _rms(x_ref[...], nw_ref[...])
    o_ref[...] = _dot_t(xn, w_ref[...])


def _attn_body(qkv_ref, o_ref, *, bq, hd, nh, d):
    i = pl.program_id(0)
    rows = i * bq + jax.lax.broadcasted_iota(jnp.int32, (bq, bq), 0)
    for h in range(nh):
        q = qkv_ref[pl.ds(i * bq, bq), h * hd:(h + 1) * hd]

        def inner(j, carry):
            m, l, acc = carry
            k = qkv_ref[pl.ds(j * bq, bq), d + h * hd:d + (h + 1) * hd]
            v = qkv_ref[pl.ds(j * bq, bq), 2 * d + h * hd:2 * d + (h + 1) * hd]
            s = _dot_t(q, k) * (1.0 / math.sqrt(hd))
            cols = j * bq + jax.lax.broadcasted_iota(jnp.int32, (bq, bq), 1)
            s = jnp.where(cols <= rows, s, _NEG)
            mj = jnp.max(s, axis=-1, keepdims=True)
            m_new = jnp.maximum(m, mj)
            p = jnp.exp(s - m_new)
            scale = jnp.exp(m - m_new)
            l_new = l * scale + jnp.sum(p, axis=-1, keepdims=True)
            acc_new = acc * scale + _dot3(p, v)
            return m_new, l_new, acc_new

        m0 = jnp.full((bq, 1), _NEG, jnp.float32)
        l0 = jnp.zeros((bq, 1), jnp.float32)
        a0 = jnp.zeros((bq, hd), jnp.float32)
        m, l, acc = jax.lax.fori_loop(0, i + 1, inner, (m0, l0, a0))
        o_ref[:, h * hd:(h + 1) * hd] = acc / l


def _proj_body(attn_ref, x_ref, ow_ref, nw_ref, wsg_ref, wsu_ref, wsd_ref,
               xres_ref, xn2_ref, shared_ref):
    a = _dot_t(attn_ref[...], ow_ref[...])
    xr = x_ref[...] + a
    xres_ref[...] = xr
    xn = _rms(xr, nw_ref[...])
    xnb = xn.astype(jnp.bfloat16)
    xn2_ref[...] = xnb
    g = jnp.dot(xnb, wsg_ref[...], preferred_element_type=jnp.float32)
    u = jnp.dot(xnb, wsu_ref[...], preferred_element_type=jnp.float32)
    hs = (g * jax.nn.sigmoid(g) * u).astype(jnp.bfloat16)
    shared_ref[...] = jnp.dot(hs, wsd_ref[...], preferred_element_type=jnp.float32)


def _router_body(xres_ref, nw_ref, rwt_ref, cw_ref, aux_ref, *, ne, coeff):
    xn = _rms(xres_ref[...], nw_ref[...])
    logits = _dot_t(xn, rwt_ref[...])
    lm = jnp.max(logits, axis=-1, keepdims=True)
    ex = jnp.exp(logits - lm)
    probs = ex / jnp.sum(ex, axis=-1, keepdims=True)
    idx = jax.lax.broadcasted_iota(jnp.int32, probs.shape, 1)
    m1 = jnp.max(probs, axis=-1, keepdims=True)
    i1 = jnp.min(jnp.where(probs == m1, idx, ne), axis=-1, keepdims=True)
    oh1 = (idx == i1)
    p2 = jnp.where(oh1, _NEG, probs)
    m2 = jnp.max(p2, axis=-1, keepdims=True)
    i2 = jnp.min(jnp.where(p2 == m2, idx, ne), axis=-1, keepdims=True)
    oh2 = (idx == i2)
    denom = m1 + m2
    cw_ref[...] = (jnp.where(oh1, m1, 0.0) + jnp.where(oh2, m2, 0.0)) / denom
    frac = jnp.mean((oh1 | oh2).astype(jnp.float32), axis=0, keepdims=True)
    pmean = jnp.mean(probs, axis=0, keepdims=True)
    aux_ref[...] = jnp.sum(frac * pmean).reshape(1, 1) * (coeff * ne)


def _moe_body(xn2_ref, wg_ref, wu_ref, wd_ref, cw_ref, xres_ref, shared_ref,
              o_ref, *, ne):
    e = pl.program_id(0)
    xb = xn2_ref[...]
    g = jnp.dot(xb, wg_ref[0], preferred_element_type=jnp.float32)
    u = jnp.dot(xb, wu_ref[0], preferred_element_type=jnp.float32)
    hh = (g * jax.nn.sigmoid(g) * u).astype(jnp.bfloat16)
    ye = jnp.dot(hh, wd_ref[0], preferred_element_type=jnp.float32)
    cwb = cw_ref[...]
    lane = jax.lax.broadcasted_iota(jnp.int32, cwb.shape, 1)
    w_col = jnp.sum(jnp.where(lane == e, cwb, 0.0), axis=-1, keepdims=True)
    contrib = w_col * ye

    @pl.when(e == 0)
    def _():
        o_ref[...] = xres_ref[...] + shared_ref[...] + contrib

    @pl.when(e > 0)
    def _():
        o_ref[...] += contrib


def kernel(x, attn_norm_w, qkv_w, out_w, ffn_norm_w, router_w, w_gate, w_up,
           w_down, ws_gate, ws_up, ws_down):
    B, T, D = x.shape
    E, _, F = w_gate.shape
    H = 16
    hd = D // H
    BT = min(256, T)
    BM = min(512, T)
    x2 = x.reshape(T, D)
    bf = jnp.bfloat16

    anw = attn_norm_w.reshape(1, D)
    fnw = ffn_norm_w.reshape(1, D)

    qkv = pl.pallas_call(
        _qkv_body,
        grid=(T // BT,),
        in_specs=[
            pl.BlockSpec((BT, D), lambda i: (i, 0)),
            pl.BlockSpec((1, D), lambda i: (0, 0)),
            pl.BlockSpec((3 * D, D), lambda i: (0, 0)),
        ],
        out_specs=pl.BlockSpec((BT, 3 * D), lambda i: (i, 0)),
        out_shape=jax.ShapeDtypeStruct((T, 3 * D), jnp.float32),
        compiler_params=pltpu.CompilerParams(
            dimension_semantics=("arbitrary",)),
    )(x2, anw, qkv_w)

    BQ = min(512, T)
    attn = pl.pallas_call(
        functools.partial(_attn_body, bq=BQ, hd=hd, nh=H, d=D),
        grid=(T // BQ,),
        in_specs=[
            pl.BlockSpec((T, 3 * D), lambda i: (0, 0)),
        ],
        out_specs=pl.BlockSpec((BQ, D), lambda i: (i, 0)),
        out_shape=jax.ShapeDtypeStruct((T, D), jnp.float32),
        compiler_params=pltpu.CompilerParams(
            dimension_semantics=("arbitrary",)),
    )(qkv)

    xres, xn2, shared = pl.pallas_call(
        _proj_body,
        grid=(T // BT,),
        in_specs=[
            pl.BlockSpec((BT, D), lambda i: (i, 0)),
            pl.BlockSpec((BT, D), lambda i: (i, 0)),
            pl.BlockSpec((D, D), lambda i: (0, 0)),
            pl.BlockSpec((1, D), lambda i: (0, 0)),
            pl.BlockSpec((D, F), lambda i: (0, 0)),
            pl.BlockSpec((D, F), lambda i: (0, 0)),
            pl.BlockSpec((F, D), lambda i: (0, 0)),
        ],
        out_specs=[
            pl.BlockSpec((BT, D), lambda i: (i, 0)),
            pl.BlockSpec((BT, D), lambda i: (i, 0)),
            pl.BlockSpec((BT, D), lambda i: (i, 0)),
        ],
        out_shape=[
            jax.ShapeDtypeStruct((T, D), jnp.float32),
            jax.ShapeDtypeStruct((T, D), bf),
            jax.ShapeDtypeStruct((T, D), jnp.float32),
        ],
        compiler_params=pltpu.CompilerParams(
            dimension_semantics=("arbitrary",)),
    )(attn, x2, out_w, fnw, ws_gate.astype(bf), ws_up.astype(bf),
      ws_down.astype(bf))

    cw, aux = pl.pallas_call(
        functools.partial(_router_body, ne=E, coeff=0.01),
        grid=(1,),
        in_specs=[
            pl.BlockSpec((T, D), lambda i: (0, 0)),
            pl.BlockSpec((1, D), lambda i: (0, 0)),
            pl.BlockSpec((E, D), lambda i: (0, 0)),
        ],
        out_specs=[
            pl.BlockSpec((T, E), lambda i: (0, 0)),
            pl.BlockSpec((1, 1), lambda i: (0, 0)),
        ],
        out_shape=[
            jax.ShapeDtypeStruct((T, E), jnp.float32),
            jax.ShapeDtypeStruct((1, 1), jnp.float32),
        ],
    )(xres, fnw, router_w)

    y = pl.pallas_call(
        functools.partial(_moe_body, ne=E),
        grid=(E,),
        in_specs=[
            pl.BlockSpec((T, D), lambda e: (0, 0)),
            pl.BlockSpec((1, D, F), lambda e: (e, 0, 0)),
            pl.BlockSpec((1, D, F), lambda e: (e, 0, 0)),
            pl.BlockSpec((1, F, D), lambda e: (e, 0, 0)),
            pl.BlockSpec((T, E), lambda e: (0, 0)),
            pl.BlockSpec((T, D), lambda e: (0, 0)),
            pl.BlockSpec((T, D), lambda e: (0, 0)),
        ],
        out_specs=pl.BlockSpec((T, D), lambda e: (0, 0)),
        out_shape=jax.ShapeDtypeStruct((T, D), jnp.float32),
        compiler_params=pltpu.CompilerParams(
            dimension_semantics=("arbitrary",)),
    )(xn2, w_gate.astype(bf), w_up.astype(bf), w_down.astype(bf), cw, xres,
      shared)

    return (y.reshape(B, T, D), aux[0, 0])


# speedup vs baseline: 3.1781x; 1.1068x over previous
import functools
import math

import jax
import jax.numpy as jnp
from jax.experimental import pallas as pl
from jax.experimental.pallas import tpu as pltpu

_EPS = 1e-6
_NEG = -1e30


def _dot_t(a, b):
    return jax.lax.dot_general(a, b, (((1,), (1,)), ((), ())),
                               preferred_element_type=jnp.float32)


def _dot3(a, b):
    return jnp.dot(a, b, preferred_element_type=jnp.float32)


def _rms(xf, w):
    ms = jnp.mean(xf * xf, axis=-1, keepdims=True)
    return xf / jnp.sqrt(ms + _EPS) * w


def _qkv_body(x_ref, nw_ref, w_ref, o_ref):
    xn = _rms(x_ref[...], nw_ref[...])
    o_ref[...] = _dot_t(xn, w_ref[...])


def _attn_body(qkv_ref, o_ref, *, bq, hd, nh, d):
    i = pl.program_id(0)
    rows = i * bq + jax.lax.broadcasted_iota(jnp.int32, (bq, bq), 0)
    for h in range(nh):
        q = qkv_ref[pl.ds(i * bq, bq), h * hd:(h + 1) * hd]

        def inner(j, carry):
            m, l, acc = carry
            k = qkv_ref[pl.ds(j * bq, bq), d + h * hd:d + (h + 1) * hd]
            v = qkv_ref[pl.ds(j * bq, bq), 2 * d + h * hd:2 * d + (h + 1) * hd]
            s = _dot_t(q, k) * (1.0 / math.sqrt(hd))
            cols = j * bq + jax.lax.broadcasted_iota(jnp.int32, (bq, bq), 1)
            s = jnp.where(cols <= rows, s, _NEG)
            mj = jnp.max(s, axis=-1, keepdims=True)
            m_new = jnp.maximum(m, mj)
            p = jnp.exp(s - m_new)
            scale = jnp.exp(m - m_new)
            l_new = l * scale + jnp.sum(p, axis=-1, keepdims=True)
            acc_new = acc * scale + _dot3(p, v)
            return m_new, l_new, acc_new

        m0 = jnp.full((bq, 1), _NEG, jnp.float32)
        l0 = jnp.zeros((bq, 1), jnp.float32)
        a0 = jnp.zeros((bq, hd), jnp.float32)
        m, l, acc = jax.lax.fori_loop(0, i + 1, inner, (m0, l0, a0))
        o_ref[:, h * hd:(h + 1) * hd] = acc / l


def _proj_body(attn_ref, x_ref, ow_ref, nw_ref, wsg_ref, wsu_ref, wsd_ref,
               xres_ref, xn2_ref, shared_ref):
    a = _dot_t(attn_ref[...], ow_ref[...])
    xr = x_ref[...] + a
    xres_ref[...] = xr
    xn = _rms(xr, nw_ref[...])
    xnb = xn.astype(jnp.bfloat16)
    xn2_ref[...] = xnb
    g = jnp.dot(xnb, wsg_ref[...].astype(jnp.bfloat16),
                preferred_element_type=jnp.float32)
    u = jnp.dot(xnb, wsu_ref[...].astype(jnp.bfloat16),
                preferred_element_type=jnp.float32)
    hs = (g * jax.nn.sigmoid(g) * u).astype(jnp.bfloat16)
    shared_ref[...] = jnp.dot(hs, wsd_ref[...].astype(jnp.bfloat16),
                              preferred_element_type=jnp.float32)


def _router_body(xres_ref, nw_ref, rwt_ref, cw_ref, aux_ref, *, ne, coeff):
    xn = _rms(xres_ref[...], nw_ref[...])
    logits = _dot_t(xn, rwt_ref[...])
    lm = jnp.max(logits, axis=-1, keepdims=True)
    ex = jnp.exp(logits - lm)
    probs = ex / jnp.sum(ex, axis=-1, keepdims=True)
    idx = jax.lax.broadcasted_iota(jnp.int32, probs.shape, 1)
    m1 = jnp.max(probs, axis=-1, keepdims=True)
    i1 = jnp.min(jnp.where(probs == m1, idx, ne), axis=-1, keepdims=True)
    oh1 = (idx == i1)
    p2 = jnp.where(oh1, _NEG, probs)
    m2 = jnp.max(p2, axis=-1, keepdims=True)
    i2 = jnp.min(jnp.where(p2 == m2, idx, ne), axis=-1, keepdims=True)
    oh2 = (idx == i2)
    denom = m1 + m2
    cw_ref[...] = (jnp.where(oh1, m1, 0.0) + jnp.where(oh2, m2, 0.0)) / denom
    frac = jnp.mean((oh1 | oh2).astype(jnp.float32), axis=0, keepdims=True)
    pmean = jnp.mean(probs, axis=0, keepdims=True)
    aux_ref[...] = jnp.sum(frac * pmean).reshape(1, 1) * (coeff * ne)


def _moe_body(xn2_ref, wg_ref, wu_ref, wd_ref, cw_ref, xres_ref, shared_ref,
              o_ref, *, ne):
    e = pl.program_id(0)
    xb = xn2_ref[...]
    g = jnp.dot(xb, wg_ref[0].astype(jnp.bfloat16),
                preferred_element_type=jnp.float32)
    u = jnp.dot(xb, wu_ref[0].astype(jnp.bfloat16),
                preferred_element_type=jnp.float32)
    hh = (g * jax.nn.sigmoid(g) * u).astype(jnp.bfloat16)
    ye = jnp.dot(hh, wd_ref[0].astype(jnp.bfloat16),
                 preferred_element_type=jnp.float32)
    cwb = cw_ref[...]
    lane = jax.lax.broadcasted_iota(jnp.int32, cwb.shape, 1)
    w_col = jnp.sum(jnp.where(lane == e, cwb, 0.0), axis=-1, keepdims=True)
    contrib = w_col * ye

    @pl.when(e == 0)
    def _():
        o_ref[...] = xres_ref[...] + shared_ref[...] + contrib

    @pl.when(e > 0)
    def _():
        o_ref[...] += contrib


def kernel(x, attn_norm_w, qkv_w, out_w, ffn_norm_w, router_w, w_gate, w_up,
           w_down, ws_gate, ws_up, ws_down):
    B, T, D = x.shape
    E, _, F = w_gate.shape
    H = 16
    hd = D // H
    BT = min(256, T)
    BM = min(512, T)
    x2 = x.reshape(T, D)
    bf = jnp.bfloat16

    anw = attn_norm_w.reshape(1, D)
    fnw = ffn_norm_w.reshape(1, D)

    qkv = pl.pallas_call(
        _qkv_body,
        grid=(T // BT,),
        in_specs=[
            pl.BlockSpec((BT, D), lambda i: (i, 0)),
            pl.BlockSpec((1, D), lambda i: (0, 0)),
            pl.BlockSpec((3 * D, D), lambda i: (0, 0)),
        ],
        out_specs=pl.BlockSpec((BT, 3 * D), lambda i: (i, 0)),
        out_shape=jax.ShapeDtypeStruct((T, 3 * D), jnp.float32),
        compiler_params=pltpu.CompilerParams(
            dimension_semantics=("arbitrary",)),
    )(x2, anw, qkv_w)

    BQ = min(512, T)
    attn = pl.pallas_call(
        functools.partial(_attn_body, bq=BQ, hd=hd, nh=H, d=D),
        grid=(T // BQ,),
        in_specs=[
            pl.BlockSpec((T, 3 * D), lambda i: (0, 0)),
        ],
        out_specs=pl.BlockSpec((BQ, D), lambda i: (i, 0)),
        out_shape=jax.ShapeDtypeStruct((T, D), jnp.float32),
        compiler_params=pltpu.CompilerParams(
            dimension_semantics=("arbitrary",)),
    )(qkv)

    xres, xn2, shared = pl.pallas_call(
        _proj_body,
        grid=(T // BT,),
        in_specs=[
            pl.BlockSpec((BT, D), lambda i: (i, 0)),
            pl.BlockSpec((BT, D), lambda i: (i, 0)),
            pl.BlockSpec((D, D), lambda i: (0, 0)),
            pl.BlockSpec((1, D), lambda i: (0, 0)),
            pl.BlockSpec((D, F), lambda i: (0, 0)),
            pl.BlockSpec((D, F), lambda i: (0, 0)),
            pl.BlockSpec((F, D), lambda i: (0, 0)),
        ],
        out_specs=[
            pl.BlockSpec((BT, D), lambda i: (i, 0)),
            pl.BlockSpec((BT, D), lambda i: (i, 0)),
            pl.BlockSpec((BT, D), lambda i: (i, 0)),
        ],
        out_shape=[
            jax.ShapeDtypeStruct((T, D), jnp.float32),
            jax.ShapeDtypeStruct((T, D), bf),
            jax.ShapeDtypeStruct((T, D), jnp.float32),
        ],
        compiler_params=pltpu.CompilerParams(
            dimension_semantics=("arbitrary",)),
    )(attn, x2, out_w, fnw, ws_gate, ws_up, ws_down)

    cw, aux = pl.pallas_call(
        functools.partial(_router_body, ne=E, coeff=0.01),
        grid=(1,),
        in_specs=[
            pl.BlockSpec((T, D), lambda i: (0, 0)),
            pl.BlockSpec((1, D), lambda i: (0, 0)),
            pl.BlockSpec((E, D), lambda i: (0, 0)),
        ],
        out_specs=[
            pl.BlockSpec((T, E), lambda i: (0, 0)),
            pl.BlockSpec((1, 1), lambda i: (0, 0)),
        ],
        out_shape=[
            jax.ShapeDtypeStruct((T, E), jnp.float32),
            jax.ShapeDtypeStruct((1, 1), jnp.float32),
        ],
    )(xres, fnw, router_w)

    y = pl.pallas_call(
        functools.partial(_moe_body, ne=E),
        grid=(E,),
        in_specs=[
            pl.BlockSpec((T, D), lambda e: (0, 0)),
            pl.BlockSpec((1, D, F), lambda e: (e, 0, 0)),
            pl.BlockSpec((1, D, F), lambda e: (e, 0, 0)),
            pl.BlockSpec((1, F, D), lambda e: (e, 0, 0)),
            pl.BlockSpec((T, E), lambda e: (0, 0)),
            pl.BlockSpec((T, D), lambda e: (0, 0)),
            pl.BlockSpec((T, D), lambda e: (0, 0)),
        ],
        out_specs=pl.BlockSpec((T, D), lambda e: (0, 0)),
        out_shape=jax.ShapeDtypeStruct((T, D), jnp.float32),
        compiler_params=pltpu.CompilerParams(
            dimension_semantics=("arbitrary",)),
    )(xn2, w_gate, w_up, w_down, cw, xres, shared)

    return (y.reshape(B, T, D), aux[0, 0])


# speedup vs baseline: 3.3345x; 1.0492x over previous
import functools
import math

import jax
import jax.numpy as jnp
from jax.experimental import pallas as pl
from jax.experimental.pallas import tpu as pltpu

_EPS = 1e-6
_NEG = -1e30


def _dot_t(a, b):
    return jax.lax.dot_general(a, b, (((1,), (1,)), ((), ())),
                               preferred_element_type=jnp.float32)


def _dot3(a, b):
    return jnp.dot(a, b, preferred_element_type=jnp.float32)


def _rms(xf, w):
    ms = jnp.mean(xf * xf, axis=-1, keepdims=True)
    return xf / jnp.sqrt(ms + _EPS) * w


def _qkv_body(x_ref, nw_ref, w_ref, o_ref):
    xn = _rms(x_ref[...], nw_ref[...])
    o_ref[...] = _dot_t(xn, w_ref[...])


def _attn_body(qkv_ref, o_ref, *, bq, hd, nh, d):
    i = pl.program_id(0)
    tri = (jax.lax.broadcasted_iota(jnp.int32, (bq, bq), 1)
           <= jax.lax.broadcasted_iota(jnp.int32, (bq, bq), 0))
    ones_blk = jnp.ones((bq, 128 - hd), jnp.float32)
    inv = 1.0 / math.sqrt(hd)
    for h in range(nh):
        q = qkv_ref[pl.ds(i * bq, bq), h * hd:(h + 1) * hd]

        def step(j, m, acc, masked):
            k = qkv_ref[pl.ds(j * bq, bq), d + h * hd:d + (h + 1) * hd]
            v = qkv_ref[pl.ds(j * bq, bq), 2 * d + h * hd:2 * d + (h + 1) * hd]
            s = _dot_t(q, k) * inv
            if masked:
                s = jnp.where(tri, s, _NEG)
            mj = jnp.max(s, axis=-1, keepdims=True)
            m_new = jnp.maximum(m, mj)
            p = jnp.exp(s - m_new)
            scale = jnp.exp(m - m_new)
            v2 = jnp.concatenate([v, ones_blk], axis=1)
            return m_new, acc * scale + _dot3(p, v2)

        m0 = jnp.full((bq, 1), _NEG, jnp.float32)
        a0 = jnp.zeros((bq, 128), jnp.float32)
        m, acc = jax.lax.fori_loop(
            0, i, lambda j, c: step(j, c[0], c[1], False), (m0, a0))
        m, acc = step(i, m, acc, True)
        o_ref[:, h * hd:(h + 1) * hd] = acc[:, :hd] / acc[:, hd:hd + 1]


def _proj_body(attn_ref, x_ref, ow_ref, nw_ref, wsg_ref, wsu_ref, wsd_ref,
               xres_ref, xn2_ref, shared_ref):
    a = _dot_t(attn_ref[...], ow_ref[...])
    xr = x_ref[...] + a
    xres_ref[...] = xr
    xn = _rms(xr, nw_ref[...])
    xnb = xn.astype(jnp.bfloat16)
    xn2_ref[...] = xnb
    g = jnp.dot(xnb, wsg_ref[...].astype(jnp.bfloat16),
                preferred_element_type=jnp.float32)
    u = jnp.dot(xnb, wsu_ref[...].astype(jnp.bfloat16),
                preferred_element_type=jnp.float32)
    hs = (g * jax.nn.sigmoid(g) * u).astype(jnp.bfloat16)
    shared_ref[...] = jnp.dot(hs, wsd_ref[...].astype(jnp.bfloat16),
                              preferred_element_type=jnp.float32)


def _router_body(xres_ref, nw_ref, rwt_ref, cw_ref, aux_ref, *, ne, coeff):
    xn = _rms(xres_ref[...], nw_ref[...])
    logits = _dot_t(xn, rwt_ref[...])
    lm = jnp.max(logits, axis=-1, keepdims=True)
    ex = jnp.exp(logits - lm)
    probs = ex / jnp.sum(ex, axis=-1, keepdims=True)
    idx = jax.lax.broadcasted_iota(jnp.int32, probs.shape, 1)
    m1 = jnp.max(probs, axis=-1, keepdims=True)
    i1 = jnp.min(jnp.where(probs == m1, idx, ne), axis=-1, keepdims=True)
    oh1 = (idx == i1)
    p2 = jnp.where(oh1, _NEG, probs)
    m2 = jnp.max(p2, axis=-1, keepdims=True)
    i2 = jnp.min(jnp.where(p2 == m2, idx, ne), axis=-1, keepdims=True)
    oh2 = (idx == i2)
    denom = m1 + m2
    cw_ref[...] = (jnp.where(oh1, m1, 0.0) + jnp.where(oh2, m2, 0.0)) / denom
    frac = jnp.mean((oh1 | oh2).astype(jnp.float32), axis=0, keepdims=True)
    pmean = jnp.mean(probs, axis=0, keepdims=True)
    aux_ref[...] = jnp.sum(frac * pmean).reshape(1, 1) * (coeff * ne)


def _moe_body(xn2_ref, wg_ref, wu_ref, wd_ref, cw_ref, xres_ref, shared_ref,
              o_ref, *, ne):
    e = pl.program_id(0)
    xb = xn2_ref[...]
    g = jnp.dot(xb, wg_ref[0].astype(jnp.bfloat16),
                preferred_element_type=jnp.float32)
    u = jnp.dot(xb, wu_ref[0].astype(jnp.bfloat16),
                preferred_element_type=jnp.float32)
    hh = (g * jax.nn.sigmoid(g) * u).astype(jnp.bfloat16)
    ye = jnp.dot(hh, wd_ref[0].astype(jnp.bfloat16),
                 preferred_element_type=jnp.float32)
    cwb = cw_ref[...]
    lane = jax.lax.broadcasted_iota(jnp.int32, cwb.shape, 1)
    w_col = jnp.sum(jnp.where(lane == e, cwb, 0.0), axis=-1, keepdims=True)
    contrib = w_col * ye

    @pl.when(e == 0)
    def _():
        o_ref[...] = xres_ref[...] + shared_ref[...] + contrib

    @pl.when(e > 0)
    def _():
        o_ref[...] += contrib


def kernel(x, attn_norm_w, qkv_w, out_w, ffn_norm_w, router_w, w_gate, w_up,
           w_down, ws_gate, ws_up, ws_down):
    B, T, D = x.shape
    E, _, F = w_gate.shape
    H = 16
    hd = D // H
    BT = min(256, T)
    BM = min(512, T)
    x2 = x.reshape(T, D)
    bf = jnp.bfloat16

    anw = attn_norm_w.reshape(1, D)
    fnw = ffn_norm_w.reshape(1, D)

    qkv = pl.pallas_call(
        _qkv_body,
        grid=(T // BT,),
        in_specs=[
            pl.BlockSpec((BT, D), lambda i: (i, 0)),
            pl.BlockSpec((1, D), lambda i: (0, 0)),
            pl.BlockSpec((3 * D, D), lambda i: (0, 0)),
        ],
        out_specs=pl.BlockSpec((BT, 3 * D), lambda i: (i, 0)),
        out_shape=jax.ShapeDtypeStruct((T, 3 * D), jnp.float32),
        compiler_params=pltpu.CompilerParams(
            dimension_semantics=("arbitrary",)),
    )(x2, anw, qkv_w)

    BQ = min(512, T)
    attn = pl.pallas_call(
        functools.partial(_attn_body, bq=BQ, hd=hd, nh=H, d=D),
        grid=(T // BQ,),
        in_specs=[
            pl.BlockSpec((T, 3 * D), lambda i: (0, 0)),
        ],
        out_specs=pl.BlockSpec((BQ, D), lambda i: (i, 0)),
        out_shape=jax.ShapeDtypeStruct((T, D), jnp.float32),
        compiler_params=pltpu.CompilerParams(
            dimension_semantics=("arbitrary",)),
    )(qkv)

    xres, xn2, shared = pl.pallas_call(
        _proj_body,
        grid=(T // BT,),
        in_specs=[
            pl.BlockSpec((BT, D), lambda i: (i, 0)),
            pl.BlockSpec((BT, D), lambda i: (i, 0)),
            pl.BlockSpec((D, D), lambda i: (0, 0)),
            pl.BlockSpec((1, D), lambda i: (0, 0)),
            pl.BlockSpec((D, F), lambda i: (0, 0)),
            pl.BlockSpec((D, F), lambda i: (0, 0)),
            pl.BlockSpec((F, D), lambda i: (0, 0)),
        ],
        out_specs=[
            pl.BlockSpec((BT, D), lambda i: (i, 0)),
            pl.BlockSpec((BT, D), lambda i: (i, 0)),
            pl.BlockSpec((BT, D), lambda i: (i, 0)),
        ],
        out_shape=[
            jax.ShapeDtypeStruct((T, D), jnp.float32),
            jax.ShapeDtypeStruct((T, D), bf),
            jax.ShapeDtypeStruct((T, D), jnp.float32),
        ],
        compiler_params=pltpu.CompilerParams(
            dimension_semantics=("arbitrary",)),
    )(attn, x2, out_w, fnw, ws_gate, ws_up, ws_down)

    cw, aux = pl.pallas_call(
        functools.partial(_router_body, ne=E, coeff=0.01),
        grid=(1,),
        in_specs=[
            pl.BlockSpec((T, D), lambda i: (0, 0)),
            pl.BlockSpec((1, D), lambda i: (0, 0)),
            pl.BlockSpec((E, D), lambda i: (0, 0)),
        ],
        out_specs=[
            pl.BlockSpec((T, E), lambda i: (0, 0)),
            pl.BlockSpec((1, 1), lambda i: (0, 0)),
        ],
        out_shape=[
            jax.ShapeDtypeStruct((T, E), jnp.float32),
            jax.ShapeDtypeStruct((1, 1), jnp.float32),
        ],
    )(xres, fnw, router_w)

    y = pl.pallas_call(
        functools.partial(_moe_body, ne=E),
        grid=(E,),
        in_specs=[
            pl.BlockSpec((T, D), lambda e: (0, 0)),
            pl.BlockSpec((1, D, F), lambda e: (e, 0, 0)),
            pl.BlockSpec((1, D, F), lambda e: (e, 0, 0)),
            pl.BlockSpec((1, F, D), lambda e: (e, 0, 0)),
            pl.BlockSpec((T, E), lambda e: (0, 0)),
            pl.BlockSpec((T, D), lambda e: (0, 0)),
            pl.BlockSpec((T, D), lambda e: (0, 0)),
        ],
        out_specs=pl.BlockSpec((T, D), lambda e: (0, 0)),
        out_shape=jax.ShapeDtypeStruct((T, D), jnp.float32),
        compiler_params=pltpu.CompilerParams(
            dimension_semantics=("arbitrary",)),
    )(xn2, w_gate, w_up, w_down, cw, xres, shared)

    return (y.reshape(B, T, D), aux[0, 0])
